# baseline (speedup 1.0000x reference)
import functools
import math

import jax
import jax.numpy as jnp
from jax import lax
from jax.experimental import pallas as pl
from jax.experimental.pallas import tpu as pltpu
from jax.experimental.pallas import tpu_sc as plsc

SEQ = 200
D = 64
BATCH = 4096
B = BATCH * SEQ
NC = 2
NS = 16
NW = NC * NS
BPW = B // NW
GB = 100
SPC = 2
CHUNK = SPC * SEQ
NGATH = CHUNK // GB
NCHUNK = BPW // CHUNK
XROWS = B // GB
NBUF = 4
BN = 128
TPP = SEQ // 2


def _pe_body(out_ref):
    pos = lax.broadcasted_iota(jnp.int32, (SEQ, D), 0).astype(jnp.float32)
    col = lax.broadcasted_iota(jnp.int32, (SEQ, D), 1)
    k = (col // 2) * 2
    angle = pos * jnp.exp(k.astype(jnp.float32) * (-math.log(10000.0) / D))
    out_ref[...] = jnp.where(col % 2 == 0, jnp.sin(angle), jnp.cos(angle))


_pe_table = pl.pallas_call(
    _pe_body, out_shape=jax.ShapeDtypeStruct((SEQ, D), jnp.float32))


ROWS2 = B // 2
RB = 3200


def _add_body(g_ref, pep_ref, out_ref):
    out_ref[...] = g_ref[...] + jnp.tile(pep_ref[...], (RB // TPP, 1))


_add_pe = pl.pallas_call(
    _add_body,
    grid=(ROWS2 // RB,),
    in_specs=[
        pl.BlockSpec((RB, 128), lambda j: (j, 0)),
        pl.BlockSpec((TPP, 128), lambda j: (0, 0)),
    ],
    out_specs=pl.BlockSpec((RB, 128), lambda j: (j, 0)),
    out_shape=jax.ShapeDtypeStruct((ROWS2, 128), jnp.float32),
)


@functools.partial(
    pl.kernel,
    out_type=jax.ShapeDtypeStruct((BATCH, SEQ, D), jnp.float32),
    mesh=plsc.VectorSubcoreMesh(core_axis_name="c", subcore_axis_name="s"),
    scratch_types=(
        [pltpu.VMEM((NGATH, GB), jnp.int32) for _ in range(NBUF)]
        + [pltpu.VMEM((SPC, SEQ, D), jnp.float32) for _ in range(NBUF)]
        + [pltpu.SemaphoreType.DMA] * (3 * NBUF)
    ),
    compiler_params=pltpu.CompilerParams(use_tc_tiling_on_sc=False),
)
def _sc_lookup(table_hbm, x_hbm, out_hbm, *scratch):
    idxs = scratch[0:NBUF]
    rows = scratch[NBUF:2 * NBUF]
    isems = scratch[2 * NBUF:3 * NBUF]
    gsems = scratch[3 * NBUF:4 * NBUF]
    ssems = scratch[4 * NBUF:5 * NBUF]

    wid = lax.axis_index("s") * NC + lax.axis_index("c")

    def stage_idx(c, b):
        rowbase = pl.multiple_of(wid * (BPW // GB) + c * NGATH, NGATH)
        pltpu.async_copy(x_hbm.at[pl.ds(rowbase, NGATH)], idxs[b], isems[b])

    def wait_idx(b):
        pltpu.make_async_copy(
            x_hbm.at[pl.ds(0, NGATH)], idxs[b], isems[b]).wait()

    def fire_gathers(b):
        for j in range(NGATH):
            pltpu.async_copy(
                table_hbm.at[idxs[b].at[j]],
                rows[b].at[j // SPC, pl.ds((j % SPC) * GB, GB)],
                gsems[b],
            )

    def wait_gathers(b):
        for _ in range(NGATH):
            pltpu.make_async_copy(
                table_hbm.at[pl.ds(0, GB)],
                rows[b].at[0, pl.ds(0, GB)],
                gsems[b],
            ).wait()

    def fire_scatter(c, b):
        seqbase = pl.multiple_of(wid * (BPW // SEQ) + c * SPC, SPC)
        pltpu.async_copy(rows[b], out_hbm.at[pl.ds(seqbase, SPC)], ssems[b])

    def wait_scatter(b):
        pltpu.make_async_copy(
            rows[b], out_hbm.at[pl.ds(0, SPC)], ssems[b]).wait()

    def body(cc, b, fire, stage, skip_scatter_wait=False):
        gb = (b + 2) % NBUF
        if fire:
            if not skip_scatter_wait:
                wait_scatter(gb)
            wait_idx(gb)
            fire_gathers(gb)
        wait_gathers(b)
        if stage:
            stage_idx(cc + NBUF, b)
        fire_scatter(cc, b)

    for c in range(NBUF):
        stage_idx(jnp.int32(c), c)
    wait_idx(0)
    fire_gathers(0)
    wait_idx(1)
    fire_gathers(1)
    body(jnp.int32(0), 0, fire=True, stage=True, skip_scatter_wait=True)
    body(jnp.int32(1), 1, fire=True, stage=True, skip_scatter_wait=True)

    @pl.loop(0, (NCHUNK - 8) // NBUF)
    def _grp(m):
        cc0 = 2 + NBUF * m
        for off in range(NBUF):
            body(cc0 + off, (2 + off) % NBUF, fire=True, stage=True)

    body(jnp.int32(NCHUNK - 6), 2, fire=True, stage=True)
    body(jnp.int32(NCHUNK - 5), 3, fire=True, stage=True)
    body(jnp.int32(NCHUNK - 4), 0, fire=True, stage=False)
    body(jnp.int32(NCHUNK - 3), 1, fire=True, stage=False)
    body(jnp.int32(NCHUNK - 2), 2, fire=False, stage=False)
    body(jnp.int32(NCHUNK - 1), 3, fire=False, stage=False)
    for b in range(NBUF):
        wait_scatter(b)


def kernel(x, table):
    pe = _pe_table()
    pep = pe.reshape(TPP, 128)
    x2d = x.reshape(XROWS, GB).astype(jnp.int32)
    g = _sc_lookup(table, x2d)
    g2 = g.reshape(ROWS2, 128)
    y = _add_pe(g2, pep)
    return y.reshape(BATCH, SEQ, D)
